# Initial kernel scaffold; baseline (speedup 1.0000x reference)
import jax
import jax.numpy as jnp
from jax import lax
from jax.experimental import pallas as pl
from jax.experimental.pallas import tpu as pltpu
from jax.experimental.pallas import tpu_sc as plsc

N = 10000
E = 320000
D = 128
DEG_W = 16
NC, NS = 2, 16
NW = NC * NS
EPW = E // NW
CHUNK = 80
NCHUNKS = EPW // CHUNK
RPS = N // NS
ZBLK = 25
LANES = 16


def _make_seg_sum(with_deg: bool):
  out_type = [jax.ShapeDtypeStruct((NC, N, D), jnp.float32)]
  scratch = [
      pltpu.VMEM((CHUNK,), jnp.int32),
      pltpu.VMEM((CHUNK,), jnp.int32),
      pltpu.VMEM((CHUNK, D), jnp.float32),
      pltpu.VMEM((ZBLK, D), jnp.float32),
      pltpu.VMEM_SHARED((N, D), jnp.float32),
      pltpu.SemaphoreType.DMA,
  ]
  if with_deg:
    out_type.append(jax.ShapeDtypeStruct((NC, N, DEG_W), jnp.float32))
    scratch += [
        pltpu.VMEM((CHUNK, DEG_W), jnp.float32),
        pltpu.VMEM((ZBLK, DEG_W), jnp.float32),
        pltpu.VMEM_SHARED((N, DEG_W), jnp.float32),
    ]

  def body(table, srcs, dsts, *rest):
    if with_deg:
      outf, outd, sidx, didx, rows, zf, accf, sem, ones, zd, accd = rest
    else:
      outf, sidx, didx, rows, zf, accf, sem = rest
    c = lax.axis_index("c")
    s = lax.axis_index("s")
    w = c * NS + s

    def zf_fill(i, carry):
      r = i // (D // LANES)
      k = i % (D // LANES)
      zf[r, pl.ds(k * LANES, LANES)] = jnp.zeros((LANES,), jnp.float32)
      return carry
    lax.fori_loop(0, ZBLK * (D // LANES), zf_fill, 0)
    if with_deg:
      def od_fill(r, carry):
        ones[r, :] = jnp.ones((DEG_W,), jnp.float32)
        zd[r % ZBLK, :] = jnp.zeros((DEG_W,), jnp.float32)
        return carry
      lax.fori_loop(0, CHUNK, od_fill, 0)

    def zacc(j, carry):
      r0 = s * RPS + j * ZBLK
      pltpu.sync_copy(zf, accf.at[pl.ds(r0, ZBLK)])
      if with_deg:
        pltpu.sync_copy(zd, accd.at[pl.ds(r0, ZBLK)])
      return carry
    lax.fori_loop(0, RPS // ZBLK, zacc, 0)
    plsc.subcore_barrier()

    ebase = w * EPW
    def step(i, carry):
      b = ebase + i * CHUNK
      pltpu.sync_copy(srcs.at[pl.ds(b, CHUNK)], sidx)
      pltpu.sync_copy(dsts.at[pl.ds(b, CHUNK)], didx)
      pltpu.async_copy(table.at[sidx], rows, sem).wait()
      pltpu.sync_copy(rows, accf.at[didx], add=True)
      if with_deg:
        pltpu.sync_copy(ones, accd.at[didx], add=True)
      return carry
    lax.fori_loop(0, NCHUNKS, step, 0)
    plsc.subcore_barrier()

    r0 = s * RPS
    pltpu.sync_copy(accf.at[pl.ds(r0, RPS)], outf.at[c, pl.ds(r0, RPS)])
    if with_deg:
      pltpu.sync_copy(accd.at[pl.ds(r0, RPS)], outd.at[c, pl.ds(r0, RPS)])

  mesh = plsc.VectorSubcoreMesh(core_axis_name="c", subcore_axis_name="s")
  return pl.kernel(body, out_type=out_type, mesh=mesh, scratch_types=scratch)


_seg_sum_deg = _make_seg_sum(True)
_seg_sum = _make_seg_sum(False)

BN = 2000


def _tc1_body(x_ref, wl_ref, wr_ref, b_ref, z_ref, r_ref):
  xb = x_ref[...]
  z_ref[...] = jnp.dot(xb, wl_ref[...], preferred_element_type=jnp.float32)
  r_ref[...] = jnp.dot(xb, wr_ref[...], preferred_element_type=jnp.float32) + b_ref[...]


_tc1 = pl.pallas_call(
    _tc1_body,
    grid=(N // BN,),
    in_specs=[
        pl.BlockSpec((BN, D), lambda i: (i, 0)),
        pl.BlockSpec((D, D), lambda i: (0, 0)),
        pl.BlockSpec((D, D), lambda i: (0, 0)),
        pl.BlockSpec((1, D), lambda i: (0, 0)),
    ],
    out_specs=[pl.BlockSpec((BN, D), lambda i: (i, 0))] * 2,
    out_shape=[jax.ShapeDtypeStruct((N, D), jnp.float32)] * 2,
)


def _tc2_body(p0, p1, d0, d1, r1_ref, wl_ref, wr_ref, b_ref, z_ref, r_ref):
  deg = (d0[...] + d1[...])[:, 0:1]
  inv = 1.0 / jnp.maximum(deg, 1.0)
  h = jax.nn.sigmoid((p0[...] + p1[...]) * inv + r1_ref[...])
  z_ref[...] = jnp.dot(h, wl_ref[...], preferred_element_type=jnp.float32)
  r_ref[...] = jnp.dot(h, wr_ref[...], preferred_element_type=jnp.float32) + b_ref[...]


_tc2 = pl.pallas_call(
    _tc2_body,
    grid=(N // BN,),
    in_specs=[
        pl.BlockSpec((BN, D), lambda i: (i, 0)),
        pl.BlockSpec((BN, D), lambda i: (i, 0)),
        pl.BlockSpec((BN, DEG_W), lambda i: (i, 0)),
        pl.BlockSpec((BN, DEG_W), lambda i: (i, 0)),
        pl.BlockSpec((BN, D), lambda i: (i, 0)),
        pl.BlockSpec((D, D), lambda i: (0, 0)),
        pl.BlockSpec((D, D), lambda i: (0, 0)),
        pl.BlockSpec((1, D), lambda i: (0, 0)),
    ],
    out_specs=[pl.BlockSpec((BN, D), lambda i: (i, 0))] * 2,
    out_shape=[jax.ShapeDtypeStruct((N, D), jnp.float32)] * 2,
)


def _tc3_body(q0, q1, d0, d1, r2_ref, out_ref):
  deg = (d0[...] + d1[...])[:, 0:1]
  inv = 1.0 / jnp.maximum(deg, 1.0)
  out_ref[...] = (q0[...] + q1[...]) * inv + r2_ref[...]


_tc3 = pl.pallas_call(
    _tc3_body,
    grid=(N // BN,),
    in_specs=[
        pl.BlockSpec((BN, D), lambda i: (i, 0)),
        pl.BlockSpec((BN, D), lambda i: (i, 0)),
        pl.BlockSpec((BN, DEG_W), lambda i: (i, 0)),
        pl.BlockSpec((BN, DEG_W), lambda i: (i, 0)),
        pl.BlockSpec((BN, D), lambda i: (i, 0)),
    ],
    out_specs=pl.BlockSpec((BN, D), lambda i: (i, 0)),
    out_shape=jax.ShapeDtypeStruct((N, D), jnp.float32),
)


def kernel(x, edge_index, W1l, W1r, b1, W2l, W2r, b2):
  src = edge_index[0]
  dst = edge_index[1]
  z1, r1 = _tc1(x, W1l, W1r, b1.reshape(1, D))
  pf, pd = _seg_sum_deg(z1, src, dst)
  z2, r2 = _tc2(pf[0], pf[1], pd[0], pd[1], r1, W2l, W2r, b2.reshape(1, D))
  (qf,) = _seg_sum(z2, src, dst)
  out = _tc3(qf[0], qf[1], pd[0], pd[1], r2)
  return out


# speedup vs baseline: 4.0056x; 4.0056x over previous
import jax
import jax.numpy as jnp
from jax import lax
from jax.experimental import pallas as pl
from jax.experimental.pallas import tpu as pltpu
from jax.experimental.pallas import tpu_sc as plsc

N = 10000
E = 320000
D = 128
NC, NS = 2, 16
NW = NC * NS
EPW = E // NW
CHUNK = 80
NCHUNKS = EPW // CHUNK
ZBLK = 16
RPS = 624
LANES = 16

_mesh = plsc.VectorSubcoreMesh(core_axis_name="c", subcore_axis_name="s")
_partial_t = [jax.ShapeDtypeStruct((NC, N, D), jnp.float32)]


def _fill(ref, nrows, value):
  def body(i, carry):
    r = i // (D // LANES)
    k = i % (D // LANES)
    ref[r, pl.ds(k * LANES, LANES)] = jnp.full((LANES,), value, jnp.float32)
    return carry
  lax.fori_loop(0, nrows * (D // LANES), body, 0)


def _seg_body(table, srcs, dsts, outf, sidx, didx, rows, zf, accf, sem):
  c = lax.axis_index("c")
  s = lax.axis_index("s")
  w = c * NS + s

  _fill(zf, ZBLK, 0.0)

  r0 = s * RPS
  nblk = jnp.where(s == NS - 1, (N - (NS - 1) * RPS) // ZBLK, RPS // ZBLK)
  def zacc(j, carry):
    pltpu.sync_copy(zf, accf.at[pl.ds(r0 + j * ZBLK, ZBLK)])
    return carry
  lax.fori_loop(0, nblk, zacc, 0)
  plsc.subcore_barrier()

  ebase = w * EPW
  def step(i, carry):
    b = ebase + i * CHUNK
    pltpu.sync_copy(srcs.at[pl.ds(b, CHUNK)], sidx)
    pltpu.sync_copy(dsts.at[pl.ds(b, CHUNK)], didx)
    pltpu.async_copy(table.at[sidx], rows, sem).wait()
    pltpu.sync_copy(rows, accf.at[didx], add=True)
    return carry
  lax.fori_loop(0, NCHUNKS, step, 0)
  plsc.subcore_barrier()

  def wout(j, carry):
    rj = r0 + j * ZBLK
    pltpu.sync_copy(accf.at[pl.ds(rj, ZBLK)], outf.at[c, pl.ds(rj, ZBLK)])
    return carry
  lax.fori_loop(0, nblk, wout, 0)


_seg_sum = pl.kernel(
    _seg_body,
    out_type=_partial_t,
    mesh=_mesh,
    scratch_types=[
        pltpu.VMEM((CHUNK,), jnp.int32),
        pltpu.VMEM((CHUNK,), jnp.int32),
        pltpu.VMEM((CHUNK, D), jnp.float32),
        pltpu.VMEM((ZBLK, D), jnp.float32),
        pltpu.VMEM_SHARED((N, D), jnp.float32),
        pltpu.SemaphoreType.DMA,
    ],
)


def _deg_body(dsts, outd, didx, ones, zf, accd):
  c = lax.axis_index("c")
  s = lax.axis_index("s")
  w = c * NS + s

  _fill(zf, ZBLK, 0.0)
  _fill(ones, CHUNK, 1.0)

  r0 = s * RPS
  nblk = jnp.where(s == NS - 1, (N - (NS - 1) * RPS) // ZBLK, RPS // ZBLK)
  def zacc(j, carry):
    pltpu.sync_copy(zf, accd.at[pl.ds(r0 + j * ZBLK, ZBLK)])
    return carry
  lax.fori_loop(0, nblk, zacc, 0)
  plsc.subcore_barrier()

  ebase = w * EPW
  def step(i, carry):
    pltpu.sync_copy(dsts.at[pl.ds(ebase + i * CHUNK, CHUNK)], didx)
    pltpu.sync_copy(ones, accd.at[didx], add=True)
    return carry
  lax.fori_loop(0, NCHUNKS, step, 0)
  plsc.subcore_barrier()

  def wout(j, carry):
    rj = r0 + j * ZBLK
    pltpu.sync_copy(accd.at[pl.ds(rj, ZBLK)], outd.at[c, pl.ds(rj, ZBLK)])
    return carry
  lax.fori_loop(0, nblk, wout, 0)


_deg_sum = pl.kernel(
    _deg_body,
    out_type=_partial_t,
    mesh=_mesh,
    scratch_types=[
        pltpu.VMEM((CHUNK,), jnp.int32),
        pltpu.VMEM((CHUNK, D), jnp.float32),
        pltpu.VMEM((ZBLK, D), jnp.float32),
        pltpu.VMEM_SHARED((N, D), jnp.float32),
    ],
)

BN = 2000


def _tc1_body(x_ref, wl_ref, wr_ref, b_ref, z_ref, r_ref):
  xb = x_ref[...]
  z_ref[...] = jnp.dot(xb, wl_ref[...], preferred_element_type=jnp.float32)
  r_ref[...] = jnp.dot(xb, wr_ref[...], preferred_element_type=jnp.float32) + b_ref[...]


_tc1 = pl.pallas_call(
    _tc1_body,
    grid=(N // BN,),
    in_specs=[
        pl.BlockSpec((BN, D), lambda i: (i, 0)),
        pl.BlockSpec((D, D), lambda i: (0, 0)),
        pl.BlockSpec((D, D), lambda i: (0, 0)),
        pl.BlockSpec((1, D), lambda i: (0, 0)),
    ],
    out_specs=[pl.BlockSpec((BN, D), lambda i: (i, 0))] * 2,
    out_shape=[jax.ShapeDtypeStruct((N, D), jnp.float32)] * 2,
)


def _tc2_body(p0, p1, d0, d1, r1_ref, wl_ref, wr_ref, b_ref, z_ref, r_ref):
  inv = 1.0 / jnp.maximum(d0[...] + d1[...], 1.0)
  h = jax.nn.sigmoid((p0[...] + p1[...]) * inv + r1_ref[...])
  z_ref[...] = jnp.dot(h, wl_ref[...], preferred_element_type=jnp.float32)
  r_ref[...] = jnp.dot(h, wr_ref[...], preferred_element_type=jnp.float32) + b_ref[...]


_tc2 = pl.pallas_call(
    _tc2_body,
    grid=(N // BN,),
    in_specs=[
        pl.BlockSpec((BN, D), lambda i: (i, 0)),
        pl.BlockSpec((BN, D), lambda i: (i, 0)),
        pl.BlockSpec((BN, D), lambda i: (i, 0)),
        pl.BlockSpec((BN, D), lambda i: (i, 0)),
        pl.BlockSpec((BN, D), lambda i: (i, 0)),
        pl.BlockSpec((D, D), lambda i: (0, 0)),
        pl.BlockSpec((D, D), lambda i: (0, 0)),
        pl.BlockSpec((1, D), lambda i: (0, 0)),
    ],
    out_specs=[pl.BlockSpec((BN, D), lambda i: (i, 0))] * 2,
    out_shape=[jax.ShapeDtypeStruct((N, D), jnp.float32)] * 2,
)


def _tc3_body(q0, q1, d0, d1, r2_ref, out_ref):
  inv = 1.0 / jnp.maximum(d0[...] + d1[...], 1.0)
  out_ref[...] = (q0[...] + q1[...]) * inv + r2_ref[...]


_tc3 = pl.pallas_call(
    _tc3_body,
    grid=(N // BN,),
    in_specs=[pl.BlockSpec((BN, D), lambda i: (i, 0))] * 5,
    out_specs=pl.BlockSpec((BN, D), lambda i: (i, 0)),
    out_shape=jax.ShapeDtypeStruct((N, D), jnp.float32),
)


def kernel(x, edge_index, W1l, W1r, b1, W2l, W2r, b2):
  src = edge_index[0]
  dst = edge_index[1]
  (degp,) = _deg_sum(dst)
  z1, r1 = _tc1(x, W1l, W1r, b1.reshape(1, D))
  (pf,) = _seg_sum(z1, src, dst)
  z2, r2 = _tc2(pf[0], pf[1], degp[0], degp[1], r1, W2l, W2r, b2.reshape(1, D))
  (qf,) = _seg_sum(z2, src, dst)
  out = _tc3(qf[0], qf[1], degp[0], degp[1], r2)
  return out


# speedup vs baseline: 6.7505x; 1.6853x over previous
import jax
import jax.numpy as jnp
from jax import lax
from jax.experimental import pallas as pl
from jax.experimental.pallas import tpu as pltpu
from jax.experimental.pallas import tpu_sc as plsc

N = 10000
E = 320000
D = 128
NC, NS = 2, 16
NW = NC * NS
EPW = E // NW
CHUNK = 80
NCHUNKS = EPW // CHUNK
ZBLK = 16
RPS = 624
LANES = 16

_mesh = plsc.VectorSubcoreMesh(core_axis_name="c", subcore_axis_name="s")
_partial_t = [jax.ShapeDtypeStruct((NC, N, D), jnp.float32)]


def _fill(ref, nrows, value):
  def body(i, carry):
    r = i // (D // LANES)
    k = i % (D // LANES)
    ref[r, pl.ds(k * LANES, LANES)] = jnp.full((LANES,), value, jnp.float32)
    return carry
  lax.fori_loop(0, nrows * (D // LANES), body, 0)


def _zero_acc(zf, acc, s, zsem):
  del zsem
  r0 = s * RPS
  nblk = jnp.where(s == NS - 1, (N - (NS - 1) * RPS) // ZBLK, RPS // ZBLK)
  def fire(j, carry):
    pltpu.sync_copy(zf, acc.at[pl.ds(r0 + j * ZBLK, ZBLK)])
    return carry
  lax.fori_loop(0, nblk, fire, 0)
  return r0, nblk


def _write_out(acc, out, c, r0, nblk, zsem):
  del zsem
  def fire(j, carry):
    rj = r0 + j * ZBLK
    pltpu.sync_copy(acc.at[pl.ds(rj, ZBLK)], out.at[c, pl.ds(rj, ZBLK)])
    return carry
  lax.fori_loop(0, nblk, fire, 0)


def _seg_body(table, srcs, dst3, outf, srcix, didx, rows, zf, accf, gsem, zsem):
  c = lax.axis_index("c")
  s = lax.axis_index("s")
  w = c * NS + s

  spre = pltpu.async_copy(srcs.at[pl.ds(w * EPW, EPW)], srcix, gsem)
  dpre = pltpu.async_copy(dst3.at[w], didx, zsem)
  _fill(zf, ZBLK, 0.0)
  spre.wait()
  dpre.wait()

  r0, nblk = _zero_acc(zf, accf, s, zsem)
  plsc.subcore_barrier()

  def gix(i):
    return srcix.at[pl.ds(i * CHUNK, CHUNK)]
  pltpu.async_copy(table.at[gix(0)], rows.at[0], gsem).wait()
  def step(i, carry):
    b = lax.rem(i, 2)
    nxt = pltpu.async_copy(table.at[gix(i + 1)], rows.at[1 - b], gsem)
    pltpu.sync_copy(rows.at[b], accf.at[didx.at[i]], add=True)
    nxt.wait()
    return carry
  lax.fori_loop(0, NCHUNKS - 1, step, 0)
  bl = (NCHUNKS - 1) % 2
  pltpu.sync_copy(rows.at[bl], accf.at[didx.at[NCHUNKS - 1]], add=True)
  plsc.subcore_barrier()

  _write_out(accf, outf, c, r0, nblk, zsem)


_seg_sum = pl.kernel(
    _seg_body,
    out_type=_partial_t,
    mesh=_mesh,
    scratch_types=[
        pltpu.VMEM((EPW,), jnp.int32),
        pltpu.VMEM((NCHUNKS, CHUNK), jnp.int32),
        pltpu.VMEM((2, CHUNK, D), jnp.float32),
        pltpu.VMEM((ZBLK, D), jnp.float32),
        pltpu.VMEM_SHARED((N, D), jnp.float32),
        pltpu.SemaphoreType.DMA,
        pltpu.SemaphoreType.DMA,
    ],
)


def _deg_body(dst3, outd, didx, ones, zf, accd, ssem, zsem):
  c = lax.axis_index("c")
  s = lax.axis_index("s")
  w = c * NS + s

  dpre = pltpu.async_copy(dst3.at[w], didx, zsem)
  _fill(zf, ZBLK, 0.0)
  _fill(ones, CHUNK, 1.0)
  dpre.wait()

  r0, nblk = _zero_acc(zf, accd, s, zsem)
  plsc.subcore_barrier()

  GRP = 5
  def step(j, carry):
    ds = [pltpu.async_copy(ones, accd.at[didx.at[GRP * j + k]], ssem, add=True)
          for k in range(GRP)]
    for dd in ds:
      dd.wait()
    return carry
  lax.fori_loop(0, NCHUNKS // GRP, step, 0)
  plsc.subcore_barrier()

  _write_out(accd, outd, c, r0, nblk, zsem)


_deg_sum = pl.kernel(
    _deg_body,
    out_type=_partial_t,
    mesh=_mesh,
    scratch_types=[
        pltpu.VMEM((NCHUNKS, CHUNK), jnp.int32),
        pltpu.VMEM((CHUNK, D), jnp.float32),
        pltpu.VMEM((ZBLK, D), jnp.float32),
        pltpu.VMEM_SHARED((N, D), jnp.float32),
        pltpu.SemaphoreType.DMA,
        pltpu.SemaphoreType.DMA,
    ],
)

BN = 2000


def _tc1_body(x_ref, wl_ref, wr_ref, b_ref, z_ref, r_ref):
  xb = x_ref[...]
  z_ref[...] = jnp.dot(xb, wl_ref[...], preferred_element_type=jnp.float32)
  r_ref[...] = jnp.dot(xb, wr_ref[...], preferred_element_type=jnp.float32) + b_ref[...]


_tc1 = pl.pallas_call(
    _tc1_body,
    grid=(N // BN,),
    in_specs=[
        pl.BlockSpec((BN, D), lambda i: (i, 0)),
        pl.BlockSpec((D, D), lambda i: (0, 0)),
        pl.BlockSpec((D, D), lambda i: (0, 0)),
        pl.BlockSpec((1, D), lambda i: (0, 0)),
    ],
    out_specs=[pl.BlockSpec((BN, D), lambda i: (i, 0))] * 2,
    out_shape=[jax.ShapeDtypeStruct((N, D), jnp.float32)] * 2,
)


def _tc2_body(p0, p1, d0, d1, r1_ref, wl_ref, wr_ref, b_ref, z_ref, r_ref):
  inv = 1.0 / jnp.maximum(d0[...] + d1[...], 1.0)
  h = jax.nn.sigmoid((p0[...] + p1[...]) * inv + r1_ref[...])
  z_ref[...] = jnp.dot(h, wl_ref[...], preferred_element_type=jnp.float32)
  r_ref[...] = jnp.dot(h, wr_ref[...], preferred_element_type=jnp.float32) + b_ref[...]


_tc2 = pl.pallas_call(
    _tc2_body,
    grid=(N // BN,),
    in_specs=[
        pl.BlockSpec((BN, D), lambda i: (i, 0)),
        pl.BlockSpec((BN, D), lambda i: (i, 0)),
        pl.BlockSpec((BN, D), lambda i: (i, 0)),
        pl.BlockSpec((BN, D), lambda i: (i, 0)),
        pl.BlockSpec((BN, D), lambda i: (i, 0)),
        pl.BlockSpec((D, D), lambda i: (0, 0)),
        pl.BlockSpec((D, D), lambda i: (0, 0)),
        pl.BlockSpec((1, D), lambda i: (0, 0)),
    ],
    out_specs=[pl.BlockSpec((BN, D), lambda i: (i, 0))] * 2,
    out_shape=[jax.ShapeDtypeStruct((N, D), jnp.float32)] * 2,
)


def _tc3_body(q0, q1, d0, d1, r2_ref, out_ref):
  inv = 1.0 / jnp.maximum(d0[...] + d1[...], 1.0)
  out_ref[...] = (q0[...] + q1[...]) * inv + r2_ref[...]


_tc3 = pl.pallas_call(
    _tc3_body,
    grid=(N // BN,),
    in_specs=[pl.BlockSpec((BN, D), lambda i: (i, 0))] * 5,
    out_specs=pl.BlockSpec((BN, D), lambda i: (i, 0)),
    out_shape=jax.ShapeDtypeStruct((N, D), jnp.float32),
)


def kernel(x, edge_index, W1l, W1r, b1, W2l, W2r, b2):
  src = edge_index[0]
  dst3 = edge_index[1].reshape(NW, NCHUNKS, CHUNK)
  (degp,) = _deg_sum(dst3)
  z1, r1 = _tc1(x, W1l, W1r, b1.reshape(1, D))
  (pf,) = _seg_sum(z1, src, dst3)
  z2, r2 = _tc2(pf[0], pf[1], degp[0], degp[1], r1, W2l, W2r, b2.reshape(1, D))
  (qf,) = _seg_sum(z2, src, dst3)
  out = _tc3(qf[0], qf[1], degp[0], degp[1], r2)
  return out


# speedup vs baseline: 7.8918x; 1.1691x over previous
import jax
import jax.numpy as jnp
from jax import lax
from jax.experimental import pallas as pl
from jax.experimental.pallas import tpu as pltpu
from jax.experimental.pallas import tpu_sc as plsc

N = 10000
E = 320000
D = 128
NC, NS = 2, 16
NW = NC * NS
EPW = E // NW
CHUNK = 80
NCHUNKS = EPW // CHUNK
ZBLK = 16
RPS = 624
LANES = 16

_mesh = plsc.VectorSubcoreMesh(core_axis_name="c", subcore_axis_name="s")
_partial_t = [jax.ShapeDtypeStruct((NC, N, D), jnp.float32)]


def _fill(ref, nrows, value):
  def body(i, carry):
    r = i // (D // LANES)
    k = i % (D // LANES)
    ref[r, pl.ds(k * LANES, LANES)] = jnp.full((LANES,), value, jnp.float32)
    return carry
  lax.fori_loop(0, nrows * (D // LANES), body, 0)


def _zero_acc(zf, acc, s, zsem):
  del zsem
  r0 = s * RPS
  nblk = jnp.where(s == NS - 1, (N - (NS - 1) * RPS) // ZBLK, RPS // ZBLK)
  def fire(j, carry):
    pltpu.sync_copy(zf, acc.at[pl.ds(r0 + j * ZBLK, ZBLK)])
    return carry
  lax.fori_loop(0, nblk, fire, 0)
  return r0, nblk


def _write_out(acc, out, c, r0, nblk, zsem):
  del zsem
  def fire(j, carry):
    rj = r0 + j * ZBLK
    pltpu.sync_copy(acc.at[pl.ds(rj, ZBLK)], out.at[c, pl.ds(rj, ZBLK)])
    return carry
  lax.fori_loop(0, nblk, fire, 0)


def _seg_body(table, srcs, dst3, outf, srcix, didx, rows, zf, accf, gsem, zsem):
  c = lax.axis_index("c")
  s = lax.axis_index("s")
  w = c * NS + s

  spre = pltpu.async_copy(srcs.at[pl.ds(w * EPW, EPW)], srcix, gsem)
  dpre = pltpu.async_copy(dst3.at[w], didx, zsem)
  _fill(zf, ZBLK, 0.0)
  spre.wait()
  dpre.wait()

  r0, nblk = _zero_acc(zf, accf, s, zsem)
  plsc.subcore_barrier()

  def gix(i):
    return srcix.at[pl.ds(i * CHUNK, CHUNK)]
  pltpu.async_copy(table.at[gix(0)], rows.at[0], gsem)
  pltpu.async_copy(table.at[gix(1)], rows.at[1], gsem)
  def step(i, carry):
    b = lax.rem(i, 2)
    pltpu.make_async_copy(table.at[gix(i)], rows.at[b], gsem).wait()
    pltpu.sync_copy(rows.at[b], accf.at[didx.at[i]], add=True)
    pltpu.async_copy(table.at[gix(i + 2)], rows.at[b], gsem)
    return carry
  lax.fori_loop(0, NCHUNKS - 2, step, 0)
  for i in (NCHUNKS - 2, NCHUNKS - 1):
    b = i % 2
    pltpu.make_async_copy(table.at[gix(i)], rows.at[b], gsem).wait()
    pltpu.sync_copy(rows.at[b], accf.at[didx.at[i]], add=True)
  plsc.subcore_barrier()

  _write_out(accf, outf, c, r0, nblk, zsem)


_seg_sum = pl.kernel(
    _seg_body,
    out_type=_partial_t,
    mesh=_mesh,
    scratch_types=[
        pltpu.VMEM((EPW,), jnp.int32),
        pltpu.VMEM((NCHUNKS, CHUNK), jnp.int32),
        pltpu.VMEM((2, CHUNK, D), jnp.float32),
        pltpu.VMEM((ZBLK, D), jnp.float32),
        pltpu.VMEM_SHARED((N, D), jnp.float32),
        pltpu.SemaphoreType.DMA,
        pltpu.SemaphoreType.DMA,
    ],
)


def _deg_body(dst3, outd, didx, ones, zf, accd, ssem, zsem):
  c = lax.axis_index("c")
  s = lax.axis_index("s")
  w = c * NS + s

  dpre = pltpu.async_copy(dst3.at[w], didx, zsem)
  _fill(zf, ZBLK, 0.0)
  _fill(ones, CHUNK, 1.0)
  dpre.wait()

  r0, nblk = _zero_acc(zf, accd, s, zsem)
  plsc.subcore_barrier()

  GRP = 5
  def step(j, carry):
    ds = [pltpu.async_copy(ones, accd.at[didx.at[GRP * j + k]], ssem, add=True)
          for k in range(GRP)]
    for dd in ds:
      dd.wait()
    return carry
  lax.fori_loop(0, NCHUNKS // GRP, step, 0)
  plsc.subcore_barrier()

  _write_out(accd, outd, c, r0, nblk, zsem)


_deg_sum = pl.kernel(
    _deg_body,
    out_type=_partial_t,
    mesh=_mesh,
    scratch_types=[
        pltpu.VMEM((NCHUNKS, CHUNK), jnp.int32),
        pltpu.VMEM((CHUNK, D), jnp.float32),
        pltpu.VMEM((ZBLK, D), jnp.float32),
        pltpu.VMEM_SHARED((N, D), jnp.float32),
        pltpu.SemaphoreType.DMA,
        pltpu.SemaphoreType.DMA,
    ],
)

BN = 2000


def _tc1_body(x_ref, wl_ref, wr_ref, b_ref, z_ref, r_ref):
  xb = x_ref[...]
  z_ref[...] = jnp.dot(xb, wl_ref[...], preferred_element_type=jnp.float32)
  r_ref[...] = jnp.dot(xb, wr_ref[...], preferred_element_type=jnp.float32) + b_ref[...]


_tc1 = pl.pallas_call(
    _tc1_body,
    grid=(N // BN,),
    in_specs=[
        pl.BlockSpec((BN, D), lambda i: (i, 0)),
        pl.BlockSpec((D, D), lambda i: (0, 0)),
        pl.BlockSpec((D, D), lambda i: (0, 0)),
        pl.BlockSpec((1, D), lambda i: (0, 0)),
    ],
    out_specs=[pl.BlockSpec((BN, D), lambda i: (i, 0))] * 2,
    out_shape=[jax.ShapeDtypeStruct((N, D), jnp.float32)] * 2,
)


def _tc2_body(p0, p1, d0, d1, r1_ref, wl_ref, wr_ref, b_ref, z_ref, r_ref):
  inv = 1.0 / jnp.maximum(d0[...] + d1[...], 1.0)
  h = jax.nn.sigmoid((p0[...] + p1[...]) * inv + r1_ref[...])
  z_ref[...] = jnp.dot(h, wl_ref[...], preferred_element_type=jnp.float32)
  r_ref[...] = jnp.dot(h, wr_ref[...], preferred_element_type=jnp.float32) + b_ref[...]


_tc2 = pl.pallas_call(
    _tc2_body,
    grid=(N // BN,),
    in_specs=[
        pl.BlockSpec((BN, D), lambda i: (i, 0)),
        pl.BlockSpec((BN, D), lambda i: (i, 0)),
        pl.BlockSpec((BN, D), lambda i: (i, 0)),
        pl.BlockSpec((BN, D), lambda i: (i, 0)),
        pl.BlockSpec((BN, D), lambda i: (i, 0)),
        pl.BlockSpec((D, D), lambda i: (0, 0)),
        pl.BlockSpec((D, D), lambda i: (0, 0)),
        pl.BlockSpec((1, D), lambda i: (0, 0)),
    ],
    out_specs=[pl.BlockSpec((BN, D), lambda i: (i, 0))] * 2,
    out_shape=[jax.ShapeDtypeStruct((N, D), jnp.float32)] * 2,
)


def _tc3_body(q0, q1, d0, d1, r2_ref, out_ref):
  inv = 1.0 / jnp.maximum(d0[...] + d1[...], 1.0)
  out_ref[...] = (q0[...] + q1[...]) * inv + r2_ref[...]


_tc3 = pl.pallas_call(
    _tc3_body,
    grid=(N // BN,),
    in_specs=[pl.BlockSpec((BN, D), lambda i: (i, 0))] * 5,
    out_specs=pl.BlockSpec((BN, D), lambda i: (i, 0)),
    out_shape=jax.ShapeDtypeStruct((N, D), jnp.float32),
)


def kernel(x, edge_index, W1l, W1r, b1, W2l, W2r, b2):
  src = edge_index[0]
  dst3 = edge_index[1].reshape(NW, NCHUNKS, CHUNK)
  (degp,) = _deg_sum(dst3)
  z1, r1 = _tc1(x, W1l, W1r, b1.reshape(1, D))
  (pf,) = _seg_sum(z1, src, dst3)
  z2, r2 = _tc2(pf[0], pf[1], degp[0], degp[1], r1, W2l, W2r, b2.reshape(1, D))
  (qf,) = _seg_sum(z2, src, dst3)
  out = _tc3(qf[0], qf[1], degp[0], degp[1], r2)
  return out


# speedup vs baseline: 9.2105x; 1.1671x over previous
import jax
import jax.numpy as jnp
from jax import lax
from jax.experimental import pallas as pl
from jax.experimental.pallas import tpu as pltpu
from jax.experimental.pallas import tpu_sc as plsc

N = 10000
E = 320000
D = 128
NC, NS = 2, 16
NW = NC * NS
EPW = E // NW
CHUNK = 80
NCHUNKS = EPW // CHUNK
ZBLK = 16
RPS = 624
LANES = 16

_mesh = plsc.VectorSubcoreMesh(core_axis_name="c", subcore_axis_name="s")
_partial_t = [jax.ShapeDtypeStruct((NC, N, D), jnp.float32)]


def _fill(ref, nrows, value):
  def body(i, carry):
    r = i // (D // LANES)
    k = i % (D // LANES)
    ref[r, pl.ds(k * LANES, LANES)] = jnp.full((LANES,), value, jnp.float32)
    return carry
  lax.fori_loop(0, nrows * (D // LANES), body, 0)


def _zero_acc(zf, acc, s, zsem):
  r0 = s * RPS
  nblk = jnp.where(s == NS - 1, (N - (NS - 1) * RPS) // ZBLK, RPS // ZBLK)
  def fire(j, carry):
    pltpu.async_copy(zf, acc.at[pl.ds(r0 + j * ZBLK, ZBLK)], zsem)
    return carry
  lax.fori_loop(0, nblk, fire, 0)
  def drain(j, carry):
    pltpu.make_async_copy(zf, acc.at[pl.ds(r0, ZBLK)], zsem).wait()
    return carry
  lax.fori_loop(0, nblk, drain, 0)
  return r0, nblk


def _write_out(acc, out, c, r0, nblk, zsem):
  def fire(j, carry):
    rj = r0 + j * ZBLK
    pltpu.async_copy(acc.at[pl.ds(rj, ZBLK)], out.at[c, pl.ds(rj, ZBLK)], zsem)
    return carry
  lax.fori_loop(0, nblk, fire, 0)
  def drain(j, carry):
    pltpu.make_async_copy(acc.at[pl.ds(r0, ZBLK)], out.at[c, pl.ds(r0, ZBLK)],
                          zsem).wait()
    return carry
  lax.fori_loop(0, nblk, drain, 0)


def _seg_body(table, srcs, dst3, outf, srcix, didx, rows, zf, accf, gsem, zsem):
  c = lax.axis_index("c")
  s = lax.axis_index("s")
  w = c * NS + s

  spre = pltpu.async_copy(srcs.at[pl.ds(w * EPW, EPW)], srcix, gsem)
  dpre = pltpu.async_copy(dst3.at[w], didx, zsem)
  _fill(zf, ZBLK, 0.0)
  spre.wait()
  dpre.wait()

  r0, nblk = _zero_acc(zf, accf, s, zsem)
  plsc.subcore_barrier()

  def gix(i):
    return srcix.at[pl.ds(i * CHUNK, CHUNK)]
  pltpu.async_copy(table.at[gix(0)], rows.at[0], gsem)
  pltpu.async_copy(table.at[gix(1)], rows.at[1], gsem)
  def step(i, carry):
    b = lax.rem(i, 2)
    pltpu.make_async_copy(table.at[gix(i)], rows.at[b], gsem).wait()
    pltpu.sync_copy(rows.at[b], accf.at[didx.at[i]], add=True)
    pltpu.async_copy(table.at[gix(i + 2)], rows.at[b], gsem)
    return carry
  lax.fori_loop(0, NCHUNKS - 2, step, 0)
  for i in (NCHUNKS - 2, NCHUNKS - 1):
    b = i % 2
    pltpu.make_async_copy(table.at[gix(i)], rows.at[b], gsem).wait()
    pltpu.sync_copy(rows.at[b], accf.at[didx.at[i]], add=True)
  plsc.subcore_barrier()

  _write_out(accf, outf, c, r0, nblk, zsem)


_seg_sum = pl.kernel(
    _seg_body,
    out_type=_partial_t,
    mesh=_mesh,
    scratch_types=[
        pltpu.VMEM((EPW,), jnp.int32),
        pltpu.VMEM((NCHUNKS, CHUNK), jnp.int32),
        pltpu.VMEM((2, CHUNK, D), jnp.float32),
        pltpu.VMEM((ZBLK, D), jnp.float32),
        pltpu.VMEM_SHARED((N, D), jnp.float32),
        pltpu.SemaphoreType.DMA,
        pltpu.SemaphoreType.DMA,
    ],
)


def _deg_body(dst3, outd, didx, ones, zf, accd, ssem, zsem):
  c = lax.axis_index("c")
  s = lax.axis_index("s")
  w = c * NS + s

  dpre = pltpu.async_copy(dst3.at[w], didx, zsem)
  _fill(zf, ZBLK, 0.0)
  _fill(ones, CHUNK, 1.0)
  dpre.wait()

  r0, nblk = _zero_acc(zf, accd, s, zsem)
  plsc.subcore_barrier()

  GRP = 5
  def step(j, carry):
    ds = [pltpu.async_copy(ones, accd.at[didx.at[GRP * j + k]], ssem, add=True)
          for k in range(GRP)]
    for dd in ds:
      dd.wait()
    return carry
  lax.fori_loop(0, NCHUNKS // GRP, step, 0)
  plsc.subcore_barrier()

  _write_out(accd, outd, c, r0, nblk, zsem)


_deg_sum = pl.kernel(
    _deg_body,
    out_type=_partial_t,
    mesh=_mesh,
    scratch_types=[
        pltpu.VMEM((NCHUNKS, CHUNK), jnp.int32),
        pltpu.VMEM((CHUNK, D), jnp.float32),
        pltpu.VMEM((ZBLK, D), jnp.float32),
        pltpu.VMEM_SHARED((N, D), jnp.float32),
        pltpu.SemaphoreType.DMA,
        pltpu.SemaphoreType.DMA,
    ],
)

BN = 2000


def _tc1_body(x_ref, wl_ref, wr_ref, b_ref, z_ref, r_ref):
  xb = x_ref[...]
  z_ref[...] = jnp.dot(xb, wl_ref[...], preferred_element_type=jnp.float32)
  r_ref[...] = jnp.dot(xb, wr_ref[...], preferred_element_type=jnp.float32) + b_ref[...]


_tc1 = pl.pallas_call(
    _tc1_body,
    grid=(N // BN,),
    in_specs=[
        pl.BlockSpec((BN, D), lambda i: (i, 0)),
        pl.BlockSpec((D, D), lambda i: (0, 0)),
        pl.BlockSpec((D, D), lambda i: (0, 0)),
        pl.BlockSpec((1, D), lambda i: (0, 0)),
    ],
    out_specs=[pl.BlockSpec((BN, D), lambda i: (i, 0))] * 2,
    out_shape=[jax.ShapeDtypeStruct((N, D), jnp.float32)] * 2,
)


def _tc2_body(p0, p1, d0, d1, r1_ref, wl_ref, wr_ref, b_ref, z_ref, r_ref):
  inv = 1.0 / jnp.maximum(d0[...] + d1[...], 1.0)
  h = jax.nn.sigmoid((p0[...] + p1[...]) * inv + r1_ref[...])
  z_ref[...] = jnp.dot(h, wl_ref[...], preferred_element_type=jnp.float32)
  r_ref[...] = jnp.dot(h, wr_ref[...], preferred_element_type=jnp.float32) + b_ref[...]


_tc2 = pl.pallas_call(
    _tc2_body,
    grid=(N // BN,),
    in_specs=[
        pl.BlockSpec((BN, D), lambda i: (i, 0)),
        pl.BlockSpec((BN, D), lambda i: (i, 0)),
        pl.BlockSpec((BN, D), lambda i: (i, 0)),
        pl.BlockSpec((BN, D), lambda i: (i, 0)),
        pl.BlockSpec((BN, D), lambda i: (i, 0)),
        pl.BlockSpec((D, D), lambda i: (0, 0)),
        pl.BlockSpec((D, D), lambda i: (0, 0)),
        pl.BlockSpec((1, D), lambda i: (0, 0)),
    ],
    out_specs=[pl.BlockSpec((BN, D), lambda i: (i, 0))] * 2,
    out_shape=[jax.ShapeDtypeStruct((N, D), jnp.float32)] * 2,
)


def _tc3_body(q0, q1, d0, d1, r2_ref, out_ref):
  inv = 1.0 / jnp.maximum(d0[...] + d1[...], 1.0)
  out_ref[...] = (q0[...] + q1[...]) * inv + r2_ref[...]


_tc3 = pl.pallas_call(
    _tc3_body,
    grid=(N // BN,),
    in_specs=[pl.BlockSpec((BN, D), lambda i: (i, 0))] * 5,
    out_specs=pl.BlockSpec((BN, D), lambda i: (i, 0)),
    out_shape=jax.ShapeDtypeStruct((N, D), jnp.float32),
)


def kernel(x, edge_index, W1l, W1r, b1, W2l, W2r, b2):
  src = edge_index[0]
  dst3 = edge_index[1].reshape(NW, NCHUNKS, CHUNK)
  (degp,) = _deg_sum(dst3)
  z1, r1 = _tc1(x, W1l, W1r, b1.reshape(1, D))
  (pf,) = _seg_sum(z1, src, dst3)
  z2, r2 = _tc2(pf[0], pf[1], degp[0], degp[1], r1, W2l, W2r, b2.reshape(1, D))
  (qf,) = _seg_sum(z2, src, dst3)
  out = _tc3(qf[0], qf[1], degp[0], degp[1], r2)
  return out


# speedup vs baseline: 9.3366x; 1.0137x over previous
import jax
import jax.numpy as jnp
from jax import lax
from jax.experimental import pallas as pl
from jax.experimental.pallas import tpu as pltpu
from jax.experimental.pallas import tpu_sc as plsc

N = 10000
E = 320000
D = 128
NC, NS = 2, 16
NW = NC * NS
EPW = E // NW
CHUNK = 80
NCHUNKS = EPW // CHUNK
ZBLK = 16
RPS = 624
LANES = 16

_mesh = plsc.VectorSubcoreMesh(core_axis_name="c", subcore_axis_name="s")
_partial_t = [jax.ShapeDtypeStruct((NC, N, D), jnp.float32)]


def _fill(ref, nrows, value):
  def body(i, carry):
    r = i // (D // LANES)
    k = i % (D // LANES)
    ref[r, pl.ds(k * LANES, LANES)] = jnp.full((LANES,), value, jnp.float32)
    return carry
  lax.fori_loop(0, nrows * (D // LANES), body, 0)


def _zero_acc(zf, acc, s, zsem):
  r0 = s * RPS
  nblk = jnp.where(s == NS - 1, (N - (NS - 1) * RPS) // ZBLK, RPS // ZBLK)
  def fire(j, carry):
    pltpu.async_copy(zf, acc.at[pl.ds(r0 + j * ZBLK, ZBLK)], zsem)
    return carry
  lax.fori_loop(0, nblk, fire, 0)
  def drain(j, carry):
    pltpu.make_async_copy(zf, acc.at[pl.ds(r0, ZBLK)], zsem).wait()
    return carry
  lax.fori_loop(0, nblk, drain, 0)
  return r0, nblk


def _write_out(acc, out, c, r0, nblk, zsem):
  def fire(j, carry):
    rj = r0 + j * ZBLK
    pltpu.async_copy(acc.at[pl.ds(rj, ZBLK)], out.at[c, pl.ds(rj, ZBLK)], zsem)
    return carry
  lax.fori_loop(0, nblk, fire, 0)
  def drain(j, carry):
    pltpu.make_async_copy(acc.at[pl.ds(r0, ZBLK)], out.at[c, pl.ds(r0, ZBLK)],
                          zsem).wait()
    return carry
  lax.fori_loop(0, nblk, drain, 0)


def _seg_edges(table, srcix, didx, rows, accf, gsem):
  def gix(i):
    return srcix.at[pl.ds(i * CHUNK, CHUNK)]
  pltpu.async_copy(table.at[gix(0)], rows.at[0], gsem)
  pltpu.async_copy(table.at[gix(1)], rows.at[1], gsem)
  def step(i, carry):
    b = lax.rem(i, 2)
    pltpu.make_async_copy(table.at[gix(i)], rows.at[b], gsem).wait()
    pltpu.sync_copy(rows.at[b], accf.at[didx.at[i]], add=True)
    pltpu.async_copy(table.at[gix(i + 2)], rows.at[b], gsem)
    return carry
  lax.fori_loop(0, NCHUNKS - 2, step, 0)
  for i in (NCHUNKS - 2, NCHUNKS - 1):
    b = i % 2
    pltpu.make_async_copy(table.at[gix(i)], rows.at[b], gsem).wait()
    pltpu.sync_copy(rows.at[b], accf.at[didx.at[i]], add=True)


def _deg_scatter(didx, ones, accd, ssem):
  GRP = 5
  def step(j, carry):
    ds = [pltpu.async_copy(ones, accd.at[didx.at[GRP * j + k]], ssem, add=True)
          for k in range(GRP)]
    for dd in ds:
      dd.wait()
    return carry
  lax.fori_loop(0, NCHUNKS // GRP, step, 0)


def _seg_body(table, srcs, dst3, outf, srcix, didx, rows, zf, accf, gsem, zsem):
  c = lax.axis_index("c")
  s = lax.axis_index("s")
  w = c * NS + s

  spre = pltpu.async_copy(srcs.at[pl.ds(w * EPW, EPW)], srcix, gsem)
  dpre = pltpu.async_copy(dst3.at[w], didx, zsem)
  _fill(zf, ZBLK, 0.0)
  spre.wait()
  dpre.wait()

  r0, nblk = _zero_acc(zf, accf, s, zsem)
  plsc.subcore_barrier()

  _seg_edges(table, srcix, didx, rows, accf, gsem)
  plsc.subcore_barrier()

  _write_out(accf, outf, c, r0, nblk, zsem)


_seg_sum = pl.kernel(
    _seg_body,
    out_type=_partial_t,
    mesh=_mesh,
    scratch_types=[
        pltpu.VMEM((EPW,), jnp.int32),
        pltpu.VMEM((NCHUNKS, CHUNK), jnp.int32),
        pltpu.VMEM((2, CHUNK, D), jnp.float32),
        pltpu.VMEM((ZBLK, D), jnp.float32),
        pltpu.VMEM_SHARED((N, D), jnp.float32),
        pltpu.SemaphoreType.DMA,
        pltpu.SemaphoreType.DMA,
    ],
)


def _seg_deg_body(table, srcs, dst3, outf, outd,
                  srcix, didx, rows, zf, accf, gsem, zsem):
  c = lax.axis_index("c")
  s = lax.axis_index("s")
  w = c * NS + s

  spre = pltpu.async_copy(srcs.at[pl.ds(w * EPW, EPW)], srcix, gsem)
  dpre = pltpu.async_copy(dst3.at[w], didx, zsem)
  _fill(zf, ZBLK, 0.0)
  _fill(rows.at[0], CHUNK, 1.0)
  spre.wait()
  dpre.wait()

  r0, nblk = _zero_acc(zf, accf, s, zsem)
  plsc.subcore_barrier()

  _deg_scatter(didx, rows.at[0], accf, gsem)
  plsc.subcore_barrier()

  _write_out(accf, outd, c, r0, nblk, zsem)
  _zero_acc(zf, accf, s, zsem)
  plsc.subcore_barrier()

  _seg_edges(table, srcix, didx, rows, accf, gsem)
  plsc.subcore_barrier()

  _write_out(accf, outf, c, r0, nblk, zsem)


_seg_deg_sum = pl.kernel(
    _seg_deg_body,
    out_type=_partial_t * 2,
    mesh=_mesh,
    scratch_types=[
        pltpu.VMEM((EPW,), jnp.int32),
        pltpu.VMEM((NCHUNKS, CHUNK), jnp.int32),
        pltpu.VMEM((2, CHUNK, D), jnp.float32),
        pltpu.VMEM((ZBLK, D), jnp.float32),
        pltpu.VMEM_SHARED((N, D), jnp.float32),
        pltpu.SemaphoreType.DMA,
        pltpu.SemaphoreType.DMA,
    ],
)

BN = 2000


def _tc1_body(x_ref, wl_ref, wr_ref, b_ref, z_ref, r_ref):
  xb = x_ref[...]
  z_ref[...] = jnp.dot(xb, wl_ref[...], preferred_element_type=jnp.float32)
  r_ref[...] = jnp.dot(xb, wr_ref[...], preferred_element_type=jnp.float32) + b_ref[...]


_tc1 = pl.pallas_call(
    _tc1_body,
    grid=(N // BN,),
    in_specs=[
        pl.BlockSpec((BN, D), lambda i: (i, 0)),
        pl.BlockSpec((D, D), lambda i: (0, 0)),
        pl.BlockSpec((D, D), lambda i: (0, 0)),
        pl.BlockSpec((1, D), lambda i: (0, 0)),
    ],
    out_specs=[pl.BlockSpec((BN, D), lambda i: (i, 0))] * 2,
    out_shape=[jax.ShapeDtypeStruct((N, D), jnp.float32)] * 2,
)


def _tc2_body(p0, p1, d0, d1, r1_ref, wl_ref, wr_ref, b_ref, z_ref, r_ref):
  inv = 1.0 / jnp.maximum(d0[...] + d1[...], 1.0)
  h = jax.nn.sigmoid((p0[...] + p1[...]) * inv + r1_ref[...])
  z_ref[...] = jnp.dot(h, wl_ref[...], preferred_element_type=jnp.float32)
  r_ref[...] = jnp.dot(h, wr_ref[...], preferred_element_type=jnp.float32) + b_ref[...]


_tc2 = pl.pallas_call(
    _tc2_body,
    grid=(N // BN,),
    in_specs=[
        pl.BlockSpec((BN, D), lambda i: (i, 0)),
        pl.BlockSpec((BN, D), lambda i: (i, 0)),
        pl.BlockSpec((BN, D), lambda i: (i, 0)),
        pl.BlockSpec((BN, D), lambda i: (i, 0)),
        pl.BlockSpec((BN, D), lambda i: (i, 0)),
        pl.BlockSpec((D, D), lambda i: (0, 0)),
        pl.BlockSpec((D, D), lambda i: (0, 0)),
        pl.BlockSpec((1, D), lambda i: (0, 0)),
    ],
    out_specs=[pl.BlockSpec((BN, D), lambda i: (i, 0))] * 2,
    out_shape=[jax.ShapeDtypeStruct((N, D), jnp.float32)] * 2,
)


def _tc3_body(q0, q1, d0, d1, r2_ref, out_ref):
  inv = 1.0 / jnp.maximum(d0[...] + d1[...], 1.0)
  out_ref[...] = (q0[...] + q1[...]) * inv + r2_ref[...]


_tc3 = pl.pallas_call(
    _tc3_body,
    grid=(N // BN,),
    in_specs=[pl.BlockSpec((BN, D), lambda i: (i, 0))] * 5,
    out_specs=pl.BlockSpec((BN, D), lambda i: (i, 0)),
    out_shape=jax.ShapeDtypeStruct((N, D), jnp.float32),
)


def kernel(x, edge_index, W1l, W1r, b1, W2l, W2r, b2):
  src = edge_index[0]
  dst3 = edge_index[1].reshape(NW, NCHUNKS, CHUNK)
  z1, r1 = _tc1(x, W1l, W1r, b1.reshape(1, D))
  pf, degp = _seg_deg_sum(z1, src, dst3)
  z2, r2 = _tc2(pf[0], pf[1], degp[0], degp[1], r1, W2l, W2r, b2.reshape(1, D))
  (qf,) = _seg_sum(z2, src, dst3)
  out = _tc3(qf[0], qf[1], degp[0], degp[1], r2)
  return out
